# initial kernel scaffold (unmeasured)
import jax
import jax.numpy as jnp
from jax import lax
from jax.experimental import pallas as pl
from jax.experimental.pallas import tpu as pltpu

N_DEV = 4


def kernel(x, w_mat):
    m_per, k = x.shape
    _, n_per = w_mat.shape
    m_half = m_per // 2

    def body(x_ref, w_ref, out_ref, bufL, bufR, bufO,
             amax_snd, amax_rcv, send_sems, recv_sems,
             a_send_sems, a_recv_sems):
        my = lax.axis_index("i")
        left = lax.rem(my + N_DEV - 1, N_DEV)
        right = lax.rem(my + 1, N_DEV)
        opp = lax.rem(my + 2, N_DEV)

        barrier_sem = pltpu.get_barrier_semaphore()
        for nbr in (left, right):
            pl.semaphore_signal(
                barrier_sem, inc=1,
                device_id=(nbr,), device_id_type=pl.DeviceIdType.MESH,
            )
        pl.semaphore_wait(barrier_sem, 2)

        s1r = pltpu.make_async_remote_copy(
            src_ref=x_ref, dst_ref=bufL,
            send_sem=send_sems.at[0], recv_sem=recv_sems.at[0],
            device_id=(right,), device_id_type=pl.DeviceIdType.MESH,
        )
        s1r.start()
        s1l = pltpu.make_async_remote_copy(
            src_ref=x_ref, dst_ref=bufR,
            send_sem=send_sems.at[1], recv_sem=recv_sems.at[1],
            device_id=(left,), device_id_type=pl.DeviceIdType.MESH,
        )
        s1l.start()

        w = w_ref[:, :]
        y = jnp.dot(x_ref[:, :], w, preferred_element_type=jnp.float32)
        out_ref[pl.ds(my * m_per, m_per), :] = y
        amax = jnp.max(jnp.abs(y))

        s1r.wait_recv()
        s2r = pltpu.make_async_remote_copy(
            src_ref=bufL.at[pl.ds(0, m_half)],
            dst_ref=bufO.at[pl.ds(0, m_half)],
            send_sem=send_sems.at[2], recv_sem=recv_sems.at[2],
            device_id=(right,), device_id_type=pl.DeviceIdType.MESH,
        )
        s2r.start()
        y = jnp.dot(bufL[:, :], w, preferred_element_type=jnp.float32)
        out_ref[pl.ds(left * m_per, m_per), :] = y
        amax = jnp.maximum(amax, jnp.max(jnp.abs(y)))

        s1l.wait_recv()
        s2l = pltpu.make_async_remote_copy(
            src_ref=bufR.at[pl.ds(m_half, m_half)],
            dst_ref=bufO.at[pl.ds(m_half, m_half)],
            send_sem=send_sems.at[3], recv_sem=recv_sems.at[3],
            device_id=(left,), device_id_type=pl.DeviceIdType.MESH,
        )
        s2l.start()
        y = jnp.dot(bufR[:, :], w, preferred_element_type=jnp.float32)
        out_ref[pl.ds(right * m_per, m_per), :] = y
        amax = jnp.maximum(amax, jnp.max(jnp.abs(y)))

        s2r.wait_recv()
        s2l.wait_recv()
        y = jnp.dot(bufO[:, :], w, preferred_element_type=jnp.float32)
        out_ref[pl.ds(opp * m_per, m_per), :] = y
        amax = jnp.maximum(amax, jnp.max(jnp.abs(y)))

        amax_snd[:, :] = jnp.full((8, 128), amax, jnp.float32)
        amax_rcv[0, :, :] = amax_snd[:, :]
        a_rdmas = []
        for d in (1, 2, 3):
            s = N_DEV - d
            r = pltpu.make_async_remote_copy(
                src_ref=amax_snd, dst_ref=amax_rcv.at[s],
                send_sem=a_send_sems.at[d - 1], recv_sem=a_recv_sems.at[s - 1],
                device_id=(lax.rem(my + d, N_DEV),),
                device_id_type=pl.DeviceIdType.MESH,
            )
            r.start()
            a_rdmas.append(r)
        for r in a_rdmas:
            r.wait_recv()

        ga = jnp.max(amax_rcv[:, :, :])
        scale = ga / 127.0
        q = jnp.clip(jnp.round(out_ref[:, :] / scale), -127.0, 127.0)
        out_ref[:, :] = q * scale

        s1r.wait_send()
        s1l.wait_send()
        s2r.wait_send()
        s2l.wait_send()
        for r in a_rdmas:
            r.wait_send()

    return pl.pallas_call(
        body,
        out_shape=jax.ShapeDtypeStruct((N_DEV * m_per, n_per), jnp.float32),
        in_specs=[
            pl.BlockSpec(memory_space=pltpu.VMEM),
            pl.BlockSpec(memory_space=pltpu.VMEM),
        ],
        out_specs=pl.BlockSpec(memory_space=pltpu.VMEM),
        scratch_shapes=[
            pltpu.VMEM((m_per, k), jnp.float32),
            pltpu.VMEM((m_per, k), jnp.float32),
            pltpu.VMEM((m_per, k), jnp.float32),
            pltpu.VMEM((8, 128), jnp.float32),
            pltpu.VMEM((N_DEV, 8, 128), jnp.float32),
            pltpu.SemaphoreType.DMA((4,)),
            pltpu.SemaphoreType.DMA((4,)),
            pltpu.SemaphoreType.DMA((3,)),
            pltpu.SemaphoreType.DMA((3,)),
        ],
        compiler_params=pltpu.CompilerParams(
            collective_id=0,
            vmem_limit_bytes=128 * 1024 * 1024,
        ),
    )(x, w_mat)


# baseline (device time: 308710 ns/iter reference)
import jax
import jax.numpy as jnp
from jax import lax
from jax.experimental import pallas as pl
from jax.experimental.pallas import tpu as pltpu

N_DEV = 4


def kernel(x, w_mat):
    m_per, k = x.shape
    _, n_per = w_mat.shape
    m_half = m_per // 2

    def body(x_ref, w_ref, out_ref, bufL, bufR, sx,
             amax_snd, amax_rcv, send_sems, recv_sems, local_sem,
             a_send_sems, a_recv_sems, cred_r, cred_l):
        my = lax.axis_index("i")
        left = lax.rem(my + N_DEV - 1, N_DEV)
        right = lax.rem(my + 1, N_DEV)
        opp = lax.rem(my + 2, N_DEV)

        barrier_sem = pltpu.get_barrier_semaphore()
        for nbr in (left, right):
            pl.semaphore_signal(
                barrier_sem, inc=1,
                device_id=(nbr,), device_id_type=pl.DeviceIdType.MESH,
            )
        pl.semaphore_wait(barrier_sem, 2)

        s1r = pltpu.make_async_remote_copy(
            src_ref=x_ref, dst_ref=bufL,
            send_sem=send_sems.at[0], recv_sem=recv_sems.at[0],
            device_id=(right,), device_id_type=pl.DeviceIdType.MESH,
        )
        s1r.start()
        s1l = pltpu.make_async_remote_copy(
            src_ref=x_ref, dst_ref=bufR,
            send_sem=send_sems.at[1], recv_sem=recv_sems.at[1],
            device_id=(left,), device_id_type=pl.DeviceIdType.MESH,
        )
        s1l.start()

        w = w_ref[:, :]
        cp0 = pltpu.make_async_copy(
            x_ref.at[pl.ds(0, m_half)], sx, local_sem)
        cp0.start()
        cp0.wait()
        y = jnp.dot(sx[:, :], w, preferred_element_type=jnp.float32)
        out_ref[pl.ds(my * m_per, m_half), :] = y
        amax = jnp.max(jnp.abs(y))
        cp1 = pltpu.make_async_copy(
            x_ref.at[pl.ds(m_half, m_half)], sx, local_sem)
        cp1.start()
        cp1.wait()
        y = jnp.dot(sx[:, :], w, preferred_element_type=jnp.float32)
        out_ref[pl.ds(my * m_per + m_half, m_half), :] = y
        amax = jnp.maximum(amax, jnp.max(jnp.abs(y)))

        s1r.wait_recv()
        y = jnp.dot(bufL[:, :], w, preferred_element_type=jnp.float32)
        out_ref[pl.ds(left * m_per, m_per), :] = y
        amax = jnp.maximum(amax, jnp.max(jnp.abs(y)))
        pl.semaphore_signal(
            cred_r, inc=1,
            device_id=(left,), device_id_type=pl.DeviceIdType.MESH,
        )
        pl.semaphore_wait(cred_r, 1)
        s2r = pltpu.make_async_remote_copy(
            src_ref=bufL.at[pl.ds(0, m_half)],
            dst_ref=bufL.at[pl.ds(m_half, m_half)],
            send_sem=send_sems.at[2], recv_sem=recv_sems.at[2],
            device_id=(right,), device_id_type=pl.DeviceIdType.MESH,
        )
        s2r.start()

        s1l.wait_recv()
        y = jnp.dot(bufR[:, :], w, preferred_element_type=jnp.float32)
        out_ref[pl.ds(right * m_per, m_per), :] = y
        amax = jnp.maximum(amax, jnp.max(jnp.abs(y)))
        pl.semaphore_signal(
            cred_l, inc=1,
            device_id=(right,), device_id_type=pl.DeviceIdType.MESH,
        )
        pl.semaphore_wait(cred_l, 1)
        s2l = pltpu.make_async_remote_copy(
            src_ref=bufR.at[pl.ds(m_half, m_half)],
            dst_ref=bufR.at[pl.ds(0, m_half)],
            send_sem=send_sems.at[3], recv_sem=recv_sems.at[3],
            device_id=(left,), device_id_type=pl.DeviceIdType.MESH,
        )
        s2l.start()

        s2r.wait_recv()
        y = jnp.dot(bufL[pl.ds(m_half, m_half), :], w,
                    preferred_element_type=jnp.float32)
        out_ref[pl.ds(opp * m_per, m_half), :] = y
        amax = jnp.maximum(amax, jnp.max(jnp.abs(y)))
        s2l.wait_recv()
        y = jnp.dot(bufR[pl.ds(0, m_half), :], w,
                    preferred_element_type=jnp.float32)
        out_ref[pl.ds(opp * m_per + m_half, m_half), :] = y
        amax = jnp.maximum(amax, jnp.max(jnp.abs(y)))

        s1r.wait_send()
        s1l.wait_send()
        s2r.wait_send()
        s2l.wait_send()

        amax_snd[:, :] = jnp.full((8, 128), amax, jnp.float32)
        amax_rcv[0, :, :] = amax_snd[:, :]
        a_rdmas = []
        for d in (1, 2, 3):
            s = N_DEV - d
            r = pltpu.make_async_remote_copy(
                src_ref=amax_snd, dst_ref=amax_rcv.at[s],
                send_sem=a_send_sems.at[d - 1],
                recv_sem=a_recv_sems.at[s - 1],
                device_id=(lax.rem(my + d, N_DEV),),
                device_id_type=pl.DeviceIdType.MESH,
            )
            r.start()
            a_rdmas.append(r)
        for r in a_rdmas:
            r.wait_recv()

        ga = jnp.max(amax_rcv[:, :, :])
        scale = ga / 127.0
        for b in range(N_DEV):
            blk = out_ref[pl.ds(b * m_per, m_per), :]
            q = jnp.clip(jnp.round(blk / scale), -127.0, 127.0)
            out_ref[pl.ds(b * m_per, m_per), :] = q * scale

        for r in a_rdmas:
            r.wait_send()

    return pl.pallas_call(
        body,
        out_shape=jax.ShapeDtypeStruct((N_DEV * m_per, n_per), jnp.float32),
        in_specs=[
            pl.BlockSpec(memory_space=pl.ANY),
            pl.BlockSpec(memory_space=pltpu.VMEM),
        ],
        out_specs=pl.BlockSpec(memory_space=pltpu.VMEM),
        scratch_shapes=[
            pltpu.VMEM((m_per, k), jnp.float32),
            pltpu.VMEM((m_per, k), jnp.float32),
            pltpu.VMEM((m_half, k), jnp.float32),
            pltpu.VMEM((8, 128), jnp.float32),
            pltpu.VMEM((N_DEV, 8, 128), jnp.float32),
            pltpu.SemaphoreType.DMA((4,)),
            pltpu.SemaphoreType.DMA((4,)),
            pltpu.SemaphoreType.DMA,
            pltpu.SemaphoreType.DMA((3,)),
            pltpu.SemaphoreType.DMA((3,)),
            pltpu.SemaphoreType.REGULAR,
            pltpu.SemaphoreType.REGULAR,
        ],
        compiler_params=pltpu.CompilerParams(
            collective_id=0,
            vmem_limit_bytes=64 * 1024 * 1024,
        ),
    )(x, w_mat)


# device time: 184862 ns/iter; 1.6699x vs baseline; 1.6699x over previous
import jax
import jax.numpy as jnp
from jax import lax
from jax.experimental import pallas as pl
from jax.experimental.pallas import tpu as pltpu

N_DEV = 4


def kernel(x, w_mat):
    m_per, k = x.shape
    _, n_per = w_mat.shape
    m_half = m_per // 2

    xb = x.astype(jnp.bfloat16)

    def body(x_ref, w_ref, out_ref, bufL, bufR,
             amax_snd, amax_rcv, send_sems, recv_sems,
             a_send_sems, a_recv_sems, cred_r, cred_l):
        my = lax.axis_index("i")
        left = lax.rem(my + N_DEV - 1, N_DEV)
        right = lax.rem(my + 1, N_DEV)
        opp = lax.rem(my + 2, N_DEV)

        barrier_sem = pltpu.get_barrier_semaphore()
        for nbr in (left, right):
            pl.semaphore_signal(
                barrier_sem, inc=1,
                device_id=(nbr,), device_id_type=pl.DeviceIdType.MESH,
            )
        pl.semaphore_wait(barrier_sem, 2)

        s1r = pltpu.make_async_remote_copy(
            src_ref=x_ref, dst_ref=bufL,
            send_sem=send_sems.at[0], recv_sem=recv_sems.at[0],
            device_id=(right,), device_id_type=pl.DeviceIdType.MESH,
        )
        s1r.start()
        s1l = pltpu.make_async_remote_copy(
            src_ref=x_ref, dst_ref=bufR,
            send_sem=send_sems.at[1], recv_sem=recv_sems.at[1],
            device_id=(left,), device_id_type=pl.DeviceIdType.MESH,
        )
        s1l.start()

        w = w_ref[:, :]
        y = jnp.dot(x_ref[:, :].astype(jnp.float32), w,
                    preferred_element_type=jnp.float32)
        out_ref[pl.ds(my * m_per, m_per), :] = y
        amax = jnp.max(jnp.abs(y))

        s1r.wait_recv()
        y = jnp.dot(bufL[:, :].astype(jnp.float32), w,
                    preferred_element_type=jnp.float32)
        out_ref[pl.ds(left * m_per, m_per), :] = y
        amax = jnp.maximum(amax, jnp.max(jnp.abs(y)))
        pl.semaphore_signal(
            cred_r, inc=1,
            device_id=(left,), device_id_type=pl.DeviceIdType.MESH,
        )
        pl.semaphore_wait(cred_r, 1)
        s2r = pltpu.make_async_remote_copy(
            src_ref=bufL.at[pl.ds(0, m_half)],
            dst_ref=bufL.at[pl.ds(m_half, m_half)],
            send_sem=send_sems.at[2], recv_sem=recv_sems.at[2],
            device_id=(right,), device_id_type=pl.DeviceIdType.MESH,
        )
        s2r.start()

        s1l.wait_recv()
        y = jnp.dot(bufR[:, :].astype(jnp.float32), w,
                    preferred_element_type=jnp.float32)
        out_ref[pl.ds(right * m_per, m_per), :] = y
        amax = jnp.maximum(amax, jnp.max(jnp.abs(y)))
        pl.semaphore_signal(
            cred_l, inc=1,
            device_id=(right,), device_id_type=pl.DeviceIdType.MESH,
        )
        pl.semaphore_wait(cred_l, 1)
        s2l = pltpu.make_async_remote_copy(
            src_ref=bufR.at[pl.ds(m_half, m_half)],
            dst_ref=bufR.at[pl.ds(0, m_half)],
            send_sem=send_sems.at[3], recv_sem=recv_sems.at[3],
            device_id=(left,), device_id_type=pl.DeviceIdType.MESH,
        )
        s2l.start()

        s2r.wait_recv()
        y = jnp.dot(bufL[pl.ds(m_half, m_half), :].astype(jnp.float32), w,
                    preferred_element_type=jnp.float32)
        out_ref[pl.ds(opp * m_per, m_half), :] = y
        amax = jnp.maximum(amax, jnp.max(jnp.abs(y)))
        s2l.wait_recv()
        y = jnp.dot(bufR[pl.ds(0, m_half), :].astype(jnp.float32), w,
                    preferred_element_type=jnp.float32)
        out_ref[pl.ds(opp * m_per + m_half, m_half), :] = y
        amax = jnp.maximum(amax, jnp.max(jnp.abs(y)))

        s1r.wait_send()
        s1l.wait_send()
        s2r.wait_send()
        s2l.wait_send()

        amax_snd[:, :] = jnp.full((8, 128), amax, jnp.float32)
        amax_rcv[0, :, :] = amax_snd[:, :]
        a_rdmas = []
        for d in (1, 2, 3):
            s = N_DEV - d
            r = pltpu.make_async_remote_copy(
                src_ref=amax_snd, dst_ref=amax_rcv.at[s],
                send_sem=a_send_sems.at[d - 1],
                recv_sem=a_recv_sems.at[s - 1],
                device_id=(lax.rem(my + d, N_DEV),),
                device_id_type=pl.DeviceIdType.MESH,
            )
            r.start()
            a_rdmas.append(r)
        for r in a_rdmas:
            r.wait_recv()

        ga = jnp.max(amax_rcv[:, :, :])
        scale = ga / 127.0
        for b in range(N_DEV):
            blk = out_ref[pl.ds(b * m_per, m_per), :]
            q = jnp.clip(jnp.round(blk / scale), -127.0, 127.0)
            out_ref[pl.ds(b * m_per, m_per), :] = q * scale

        for r in a_rdmas:
            r.wait_send()

    return pl.pallas_call(
        body,
        out_shape=jax.ShapeDtypeStruct((N_DEV * m_per, n_per), jnp.float32),
        in_specs=[
            pl.BlockSpec(memory_space=pltpu.VMEM),
            pl.BlockSpec(memory_space=pltpu.VMEM),
        ],
        out_specs=pl.BlockSpec(memory_space=pltpu.VMEM),
        scratch_shapes=[
            pltpu.VMEM((m_per, k), jnp.bfloat16),
            pltpu.VMEM((m_per, k), jnp.bfloat16),
            pltpu.VMEM((8, 128), jnp.float32),
            pltpu.VMEM((N_DEV, 8, 128), jnp.float32),
            pltpu.SemaphoreType.DMA((4,)),
            pltpu.SemaphoreType.DMA((4,)),
            pltpu.SemaphoreType.DMA((3,)),
            pltpu.SemaphoreType.DMA((3,)),
            pltpu.SemaphoreType.REGULAR,
            pltpu.SemaphoreType.REGULAR,
        ],
        compiler_params=pltpu.CompilerParams(
            collective_id=0,
            vmem_limit_bytes=64 * 1024 * 1024,
        ),
    )(xb, w_mat)
